# Initial kernel scaffold; baseline (speedup 1.0000x reference)
import jax
import jax.numpy as jnp
from jax.experimental import pallas as pl


def kernel(hidden_states, router_logits, w13, w2):
    raise NotImplementedError("write your pallas kernel here")


# speedup vs baseline: 1.1568x; 1.1568x over previous
import functools

import jax
import jax.numpy as jnp
from jax.experimental import pallas as pl

E = 8
TOPK = 2
D = 768
DFF = 3072
T = 2048
CAP = 640
FT = 1024
NF = DFF // FT


def _mlp_body(disp_ref, w13g_ref, w13u_ref, w2_ref, out_ref):
    f = pl.program_id(1)
    x = disp_ref[...].astype(jnp.bfloat16)
    wg = w13g_ref[0].astype(jnp.bfloat16)
    wu = w13u_ref[0].astype(jnp.bfloat16)
    g = jnp.dot(x, wg, preferred_element_type=jnp.float32)
    u = jnp.dot(x, wu, preferred_element_type=jnp.float32)
    act = (g * jax.nn.sigmoid(g)) * u
    w2b = w2_ref[0].astype(jnp.bfloat16)
    part = jnp.dot(act.astype(jnp.bfloat16), w2b,
                   preferred_element_type=jnp.float32)

    @pl.when(f == 0)
    def _():
        out_ref[...] = part

    @pl.when(f != 0)
    def _():
        out_ref[...] += part


@functools.partial(jax.jit, static_argnames=())
def _expert_mlp(disp, w13, w2):
    return pl.pallas_call(
        _mlp_body,
        grid=(E, NF),
        in_specs=[
            pl.BlockSpec((CAP, D), lambda e, f: (e, 0)),
            pl.BlockSpec((1, D, FT), lambda e, f: (e, 0, f)),
            pl.BlockSpec((1, D, FT), lambda e, f: (e, 0, f + NF)),
            pl.BlockSpec((1, FT, D), lambda e, f: (e, f, 0)),
        ],
        out_specs=pl.BlockSpec((CAP, D), lambda e, f: (e, 0)),
        out_shape=jax.ShapeDtypeStruct((E * CAP, D), jnp.float32),
    )(disp, w13, w13, w2)


def kernel(hidden_states, router_logits, w13, w2):
    probs = jax.nn.softmax(router_logits, axis=-1)
    topk_w, topk_ids = jax.lax.top_k(probs, TOPK)
    topk_w = topk_w / jnp.sum(topk_w, axis=-1, keepdims=True)

    e_flat = topk_ids.reshape(-1)
    w_flat = topk_w.reshape(-1)
    x_rep = jnp.repeat(hidden_states, TOPK, axis=0)

    one_hot = jax.nn.one_hot(e_flat, E, dtype=jnp.int32)
    pos = jnp.cumsum(one_hot, axis=0) - 1
    pos_flat = jnp.take_along_axis(pos, e_flat[:, None], axis=1)[:, 0]
    keep = pos_flat < CAP
    pos_safe = jnp.where(keep, pos_flat, 0)
    keep_f = keep.astype(hidden_states.dtype)

    disp = jnp.zeros((E * CAP, D), dtype=hidden_states.dtype)
    flat_idx = e_flat * CAP + pos_safe
    disp = disp.at[flat_idx].add(x_rep * keep_f[:, None])

    expert_out = _expert_mlp(disp, w13, w2)

    y = expert_out[flat_idx] * (w_flat * keep_f)[:, None]
    return y.reshape(T, TOPK, D).sum(axis=1)


# speedup vs baseline: 2.1985x; 1.9004x over previous
import functools

import jax
import jax.numpy as jnp
from jax.experimental import pallas as pl
from jax.experimental.pallas import tpu as pltpu
from jax.experimental.pallas import tpu_sc as plsc

E = 8
TOPK = 2
D = 768
DFF = 3072
T = 2048
CAP = 640
FT = 1024
NF = DFF // FT

NW = 32
TPW = T // NW
TRASH = E * CAP
DISP_ROWS = E * CAP + 8


def _dispatch_sc(hidden, dst1, dst2):
    mesh = plsc.VectorSubcoreMesh(core_axis_name="c", subcore_axis_name="s")

    @functools.partial(
        pl.kernel,
        out_type=jax.ShapeDtypeStruct((DISP_ROWS, D), jnp.float32),
        mesh=mesh,
        scratch_types=[
            pltpu.VMEM((TPW,), jnp.int32),
            pltpu.VMEM((TPW,), jnp.int32),
            pltpu.VMEM((TPW, D), jnp.float32),
            pltpu.SemaphoreType.DMA,
            pltpu.SemaphoreType.DMA,
        ],
    )
    def k(hidden_hbm, dst1_hbm, dst2_hbm, disp_hbm, idx1_v, idx2_v, rows_v,
          sem1, sem2):
        wid = jax.lax.axis_index("s") * 2 + jax.lax.axis_index("c")
        base = wid * TPW
        pltpu.sync_copy(dst1_hbm.at[pl.ds(base, TPW)], idx1_v)
        pltpu.sync_copy(dst2_hbm.at[pl.ds(base, TPW)], idx2_v)
        pltpu.sync_copy(hidden_hbm.at[pl.ds(base, TPW)], rows_v)
        c1 = pltpu.async_copy(rows_v, disp_hbm.at[idx1_v], sem1)
        c2 = pltpu.async_copy(rows_v, disp_hbm.at[idx2_v], sem2)
        c1.wait()
        c2.wait()

    return k(hidden, dst1, dst2)


def _mlp_body(disp_ref, w13g_ref, w13u_ref, w2_ref, out_ref):
    f = pl.program_id(1)
    x = disp_ref[...].astype(jnp.bfloat16)
    wg = w13g_ref[0].astype(jnp.bfloat16)
    wu = w13u_ref[0].astype(jnp.bfloat16)
    g = jnp.dot(x, wg, preferred_element_type=jnp.float32)
    u = jnp.dot(x, wu, preferred_element_type=jnp.float32)
    act = (g * jax.nn.sigmoid(g)) * u
    w2b = w2_ref[0].astype(jnp.bfloat16)
    part = jnp.dot(act.astype(jnp.bfloat16), w2b,
                   preferred_element_type=jnp.float32)

    @pl.when(f == 0)
    def _():
        out_ref[...] = part

    @pl.when(f != 0)
    def _():
        out_ref[...] += part


@functools.partial(jax.jit, static_argnames=())
def _expert_mlp(disp, w13, w2):
    return pl.pallas_call(
        _mlp_body,
        grid=(E, NF),
        in_specs=[
            pl.BlockSpec((CAP, D), lambda e, f: (e, 0)),
            pl.BlockSpec((1, D, FT), lambda e, f: (e, 0, f)),
            pl.BlockSpec((1, D, FT), lambda e, f: (e, 0, f + NF)),
            pl.BlockSpec((1, FT, D), lambda e, f: (e, f, 0)),
        ],
        out_specs=pl.BlockSpec((CAP, D), lambda e, f: (e, 0)),
        out_shape=jax.ShapeDtypeStruct((E * CAP, D), jnp.float32),
    )(disp, w13, w13, w2)


def kernel(hidden_states, router_logits, w13, w2):
    probs = jax.nn.softmax(router_logits, axis=-1)
    topk_w, topk_ids = jax.lax.top_k(probs, TOPK)
    topk_w = topk_w / jnp.sum(topk_w, axis=-1, keepdims=True)

    e_flat = topk_ids.reshape(-1)
    w_flat = topk_w.reshape(-1)

    one_hot = jax.nn.one_hot(e_flat, E, dtype=jnp.int32)
    pos = jnp.cumsum(one_hot, axis=0) - 1
    pos_flat = jnp.take_along_axis(pos, e_flat[:, None], axis=1)[:, 0]
    keep = pos_flat < CAP
    pos_safe = jnp.where(keep, pos_flat, 0)
    keep_f = keep.astype(hidden_states.dtype)

    flat_idx = e_flat * CAP + pos_safe
    dst = jnp.where(keep, flat_idx, TRASH).astype(jnp.int32)

    disp = _dispatch_sc(hidden_states, dst[0::2], dst[1::2])

    expert_out = _expert_mlp(disp, w13, w2)

    y = expert_out[flat_idx] * (w_flat * keep_f)[:, None]
    return y.reshape(T, TOPK, D).sum(axis=1)


# speedup vs baseline: 2.8107x; 1.2785x over previous
import functools

import jax
import jax.numpy as jnp
from jax.experimental import pallas as pl
from jax.experimental.pallas import tpu as pltpu
from jax.experimental.pallas import tpu_sc as plsc

E = 8
TOPK = 2
D = 768
DFF = 3072
T = 2048
CAP = 640
FT = 1024
NF = DFF // FT

NW = 32
TPW = T // NW
TRASH = E * CAP
DISP_ROWS = E * CAP + 8


def _dispatch_sc(hidden, dst1, dst2):
    mesh = plsc.VectorSubcoreMesh(core_axis_name="c", subcore_axis_name="s")

    @functools.partial(
        pl.kernel,
        out_type=jax.ShapeDtypeStruct((DISP_ROWS, D), jnp.float32),
        mesh=mesh,
        scratch_types=[
            pltpu.VMEM((TPW,), jnp.int32),
            pltpu.VMEM((TPW,), jnp.int32),
            pltpu.VMEM((TPW, D), jnp.float32),
            pltpu.SemaphoreType.DMA,
            pltpu.SemaphoreType.DMA,
        ],
    )
    def k(hidden_hbm, dst1_hbm, dst2_hbm, disp_hbm, idx1_v, idx2_v, rows_v,
          sem1, sem2):
        wid = jax.lax.axis_index("s") * 2 + jax.lax.axis_index("c")
        base = wid * TPW
        pltpu.sync_copy(dst1_hbm.at[pl.ds(base, TPW)], idx1_v)
        pltpu.sync_copy(dst2_hbm.at[pl.ds(base, TPW)], idx2_v)
        pltpu.sync_copy(hidden_hbm.at[pl.ds(base, TPW)], rows_v)
        c1 = pltpu.async_copy(rows_v, disp_hbm.at[idx1_v], sem1)
        c2 = pltpu.async_copy(rows_v, disp_hbm.at[idx2_v], sem2)
        c1.wait()
        c2.wait()

    return k(hidden, dst1, dst2)


def _combine_gather_sc(eo, cidx1, cidx2):
    mesh = plsc.VectorSubcoreMesh(core_axis_name="c", subcore_axis_name="s")

    @functools.partial(
        pl.kernel,
        out_type=(jax.ShapeDtypeStruct((T, D), jnp.float32),
                  jax.ShapeDtypeStruct((T, D), jnp.float32)),
        mesh=mesh,
        scratch_types=[
            pltpu.VMEM((TPW,), jnp.int32),
            pltpu.VMEM((TPW,), jnp.int32),
            pltpu.VMEM((TPW, D), jnp.float32),
            pltpu.VMEM((TPW, D), jnp.float32),
            pltpu.SemaphoreType.DMA,
            pltpu.SemaphoreType.DMA,
        ],
    )
    def k(eo_hbm, c1_hbm, c2_hbm, a_hbm, b_hbm, i1_v, i2_v, a_v, b_v, s1, s2):
        wid = jax.lax.axis_index("s") * 2 + jax.lax.axis_index("c")
        base = wid * TPW
        pltpu.sync_copy(c1_hbm.at[pl.ds(base, TPW)], i1_v)
        pltpu.sync_copy(c2_hbm.at[pl.ds(base, TPW)], i2_v)
        g1 = pltpu.async_copy(eo_hbm.at[i1_v], a_v, s1)
        g2 = pltpu.async_copy(eo_hbm.at[i2_v], b_v, s2)
        g1.wait()
        g2.wait()
        pltpu.sync_copy(a_v, a_hbm.at[pl.ds(base, TPW)])
        pltpu.sync_copy(b_v, b_hbm.at[pl.ds(base, TPW)])

    return k(eo, cidx1, cidx2)


RT = 512


def _weighted_sum_body(a_ref, b_ref, w1_ref, w2_ref, o_ref):
    o_ref[...] = a_ref[...] * w1_ref[...] + b_ref[...] * w2_ref[...]


def _weighted_sum(a, b, w1, w2):
    return pl.pallas_call(
        _weighted_sum_body,
        grid=(T // RT,),
        in_specs=[
            pl.BlockSpec((RT, D), lambda i: (i, 0)),
            pl.BlockSpec((RT, D), lambda i: (i, 0)),
            pl.BlockSpec((RT, 1), lambda i: (i, 0)),
            pl.BlockSpec((RT, 1), lambda i: (i, 0)),
        ],
        out_specs=pl.BlockSpec((RT, D), lambda i: (i, 0)),
        out_shape=jax.ShapeDtypeStruct((T, D), jnp.float32),
    )(a, b, w1, w2)


def _mlp_body(disp_ref, w13g_ref, w13u_ref, w2_ref, out_ref):
    f = pl.program_id(1)
    x = disp_ref[...].astype(jnp.bfloat16)
    wg = w13g_ref[0].astype(jnp.bfloat16)
    wu = w13u_ref[0].astype(jnp.bfloat16)
    g = jnp.dot(x, wg, preferred_element_type=jnp.float32)
    u = jnp.dot(x, wu, preferred_element_type=jnp.float32)
    act = (g * jax.nn.sigmoid(g)) * u
    w2b = w2_ref[0].astype(jnp.bfloat16)
    part = jnp.dot(act.astype(jnp.bfloat16), w2b,
                   preferred_element_type=jnp.float32)

    @pl.when(f == 0)
    def _():
        out_ref[...] = part

    @pl.when(f != 0)
    def _():
        out_ref[...] += part


@functools.partial(jax.jit, static_argnames=())
def _expert_mlp(disp, w13, w2):
    return pl.pallas_call(
        _mlp_body,
        grid=(E, NF),
        in_specs=[
            pl.BlockSpec((CAP, D), lambda e, f: (e, 0)),
            pl.BlockSpec((1, D, FT), lambda e, f: (e, 0, f)),
            pl.BlockSpec((1, D, FT), lambda e, f: (e, 0, f + NF)),
            pl.BlockSpec((1, FT, D), lambda e, f: (e, f, 0)),
        ],
        out_specs=pl.BlockSpec((CAP, D), lambda e, f: (e, 0)),
        out_shape=jax.ShapeDtypeStruct((E * CAP, D), jnp.float32),
    )(disp, w13, w13, w2)


def kernel(hidden_states, router_logits, w13, w2):
    probs = jax.nn.softmax(router_logits, axis=-1)
    topk_w, topk_ids = jax.lax.top_k(probs, TOPK)
    topk_w = topk_w / jnp.sum(topk_w, axis=-1, keepdims=True)

    e_flat = topk_ids.reshape(-1)
    w_flat = topk_w.reshape(-1)

    one_hot = jax.nn.one_hot(e_flat, E, dtype=jnp.int32)
    pos = jnp.cumsum(one_hot, axis=0) - 1
    pos_flat = jnp.take_along_axis(pos, e_flat[:, None], axis=1)[:, 0]
    keep = pos_flat < CAP
    pos_safe = jnp.where(keep, pos_flat, 0)
    keep_f = keep.astype(hidden_states.dtype)

    flat_idx = e_flat * CAP + pos_safe
    dst = jnp.where(keep, flat_idx, TRASH).astype(jnp.int32)

    disp = _dispatch_sc(hidden_states, dst[0::2], dst[1::2])

    expert_out = _expert_mlp(disp, w13, w2)

    cidx = flat_idx.astype(jnp.int32)
    a, b = _combine_gather_sc(expert_out, cidx[0::2], cidx[1::2])
    wk = (w_flat * keep_f).reshape(T, TOPK)
    return _weighted_sum(a, b, wk[:, 0:1], wk[:, 1:2])
